# baseline (speedup 1.0000x reference)
import jax
import jax.numpy as jnp
from jax import lax
from jax.experimental import pallas as pl
from jax.experimental.pallas import tpu as pltpu
from jax.experimental.pallas import tpu_sc as plsc

N_ROWS = 32 * 1024
K = 8192
D = 256
COMMIT = 0.25
H = 2

BR = 4096
BK = 2048
NK = K // BK

NC, NS = 2, 16
NW = NC * NS
CHUNK = 128


def _make_argmin_body(nr):
    def _argmin_body(x_ref, en_ref, idx_ref, loss_ref,
                     min_ref, arg_ref, acc_ref):
        r = pl.program_id(0)
        k = pl.program_id(1)

        x = x_ref[...]
        en = en_ref[...]
        mmn = jnp.dot(x, en, preferred_element_type=jnp.float32)
        x2 = jnp.sum(x * x, axis=1, keepdims=True)
        e2 = 0.25 * jnp.sum(en * en, axis=0, keepdims=True)
        dist = x2 + mmn + e2

        lmin = jnp.min(dist, axis=1, keepdims=True)
        iota = lax.broadcasted_iota(jnp.int32, (BR, BK), 1) + k * BK
        larg = jnp.min(jnp.where(dist == lmin, iota, jnp.int32(2**30)),
                       axis=1, keepdims=True)

        @pl.when(k == 0)
        def _init():
            min_ref[...] = lmin
            arg_ref[...] = larg

        @pl.when(k > 0)
        def _update():
            better = lmin < min_ref[...]
            arg_ref[...] = jnp.where(better, larg, arg_ref[...])
            min_ref[...] = jnp.where(better, lmin, min_ref[...])

        @pl.when(jnp.logical_and(r == 0, k == 0))
        def _zero_acc():
            acc_ref[0] = 0.0

        @pl.when(k == NK - 1)
        def _finish_row_block():
            idx_ref[...] = arg_ref[...]
            acc_ref[0] += jnp.sum(min_ref[...])

        @pl.when(jnp.logical_and(r == nr - 1, k == NK - 1))
        def _emit_loss():
            loss_ref[...] = jnp.full(
                (1, 1), acc_ref[0] * (COMMIT / (N_ROWS * D)),
                dtype=jnp.float32)

    return _argmin_body


def _tc_argmin(flat_x, emb_t):
    nr = flat_x.shape[0] // BR
    return pl.pallas_call(
        _make_argmin_body(nr),
        grid=(nr, NK),
        in_specs=[
            pl.BlockSpec((BR, D), lambda r, k: (r, 0)),
            pl.BlockSpec((D, BK), lambda r, k: (0, k)),
        ],
        out_specs=[
            pl.BlockSpec((BR, 1), lambda r, k: (r, 0)),
            pl.BlockSpec((1, 1), lambda r, k: (0, 0)),
        ],
        out_shape=[
            jax.ShapeDtypeStruct((flat_x.shape[0], 1), jnp.int32),
            jax.ShapeDtypeStruct((1, 1), jnp.float32),
        ],
        scratch_shapes=[
            pltpu.VMEM((BR, 1), jnp.float32),
            pltpu.VMEM((BR, 1), jnp.int32),
            pltpu.SMEM((1,), jnp.float32),
        ],
        compiler_params=pltpu.CompilerParams(
            dimension_semantics=("arbitrary", "arbitrary"),
        ),
    )(flat_x, emb_t)


def _make_sc_body(nchunk, rows_per_w):
    def _sc_body(idx_hbm, emb_hbm, q_out, counts_out,
                 idx_v, rows_v, ones_v, zeros_v, counts_sh, sem0, sem1):
        cid = lax.axis_index("c")
        sid = lax.axis_index("s")
        wid = sid * NC + cid
        base = wid * rows_per_w

        pltpu.sync_copy(idx_hbm.at[pl.ds(wid * nchunk, nchunk)], idx_v)

        def _fill(i, _):
            ones_v[pl.ds(i * 16, 16)] = jnp.ones((16,), jnp.float32)
            zeros_v[pl.ds(i * 16, 16)] = jnp.zeros((16,), jnp.float32)
            return 0
        lax.fori_loop(0, 32, _fill, 0)

        pltpu.sync_copy(zeros_v, counts_sh.at[pl.ds(sid * (K // NS), K // NS)])
        plsc.subcore_barrier()

        sems = (sem0, sem1)

        def _start(j, buf):
            return pltpu.make_async_copy(
                emb_hbm.at[idx_v.at[j]], rows_v.at[buf], sems[buf])

        _start(0, 0).start()
        _start(1, 1).start()
        for j in range(nchunk):
            buf = j % 2
            _start(j, buf).wait()
            pltpu.sync_copy(rows_v.at[buf],
                            q_out.at[pl.ds(base + j * CHUNK, CHUNK)])
            pltpu.sync_copy(ones_v.at[pl.ds(0, CHUNK)],
                            counts_sh.at[idx_v.at[j]], add=True)
            if j + 2 < nchunk:
                _start(j + 2, buf).start()

        plsc.subcore_barrier()

        @pl.when(sid == 0)
        def _emit_counts():
            pltpu.sync_copy(counts_sh, counts_out.at[cid])

    return _sc_body


def _sc_gather_bincount(idx2d, embedding):
    n_rows = idx2d.shape[0] * CHUNK
    rows_per_w = n_rows // NW
    nchunk = rows_per_w // CHUNK
    mesh = plsc.VectorSubcoreMesh(core_axis_name="c", subcore_axis_name="s")
    kern = pl.kernel(
        _make_sc_body(nchunk, rows_per_w),
        out_type=[
            jax.ShapeDtypeStruct((n_rows, D), jnp.float32),
            jax.ShapeDtypeStruct((NC, K), jnp.float32),
        ],
        mesh=mesh,
        scratch_types=[
            pltpu.VMEM((nchunk, CHUNK), jnp.int32),
            pltpu.VMEM((2, CHUNK, D), jnp.float32),
            pltpu.VMEM((512, ), jnp.float32),
            pltpu.VMEM((512, ), jnp.float32),
            pltpu.VMEM_SHARED((K,), jnp.float32),
            pltpu.SemaphoreType.DMA,
            pltpu.SemaphoreType.DMA,
        ],
    )
    return kern(idx2d, embedding)


def _finalize_body(c_ref, l_ref, perp_ref, loss_ref):
    counts = jnp.sum(c_ref[...], axis=0, keepdims=True)
    total = jnp.sum(counts)
    probs = counts / (total + 1e-10)
    ent = jnp.sum(probs * jnp.log(probs + 1e-10))
    perp_ref[...] = jnp.full((1, 1), jnp.exp(-ent), dtype=jnp.float32)
    loss_ref[...] = jnp.full((1, 1), jnp.sum(l_ref[...]), dtype=jnp.float32)


def _tc_finalize(counts_all, loss_parts):
    return pl.pallas_call(
        _finalize_body,
        out_shape=[
            jax.ShapeDtypeStruct((1, 1), jnp.float32),
            jax.ShapeDtypeStruct((1, 1), jnp.float32),
        ],
    )(counts_all, loss_parts)


@jax.jit
def kernel(inputs, embedding):
    flat = inputs.reshape(N_ROWS, D)
    emb_t_neg2 = -2.0 * embedding.T

    nh = N_ROWS // H
    q_halves, counts_parts, loss_parts, idx_halves = [], [], [], []
    for h in range(H):
        idx2d, loss2d = _tc_argmin(flat[h * nh:(h + 1) * nh], emb_t_neg2)
        idx_rows = idx2d.reshape(nh // CHUNK, CHUNK)
        q_flat, counts2 = _sc_gather_bincount(idx_rows, embedding)
        q_halves.append(q_flat)
        counts_parts.append(counts2)
        loss_parts.append(loss2d)
        idx_halves.append(idx2d)

    counts_all = jnp.concatenate(counts_parts, axis=0)
    loss_stack = jnp.concatenate(loss_parts, axis=0)
    perp2d, loss2d = _tc_finalize(counts_all, loss_stack)

    quantized_st = jnp.concatenate(q_halves, axis=0).reshape(inputs.shape)
    indices = jnp.concatenate(idx_halves, axis=0).reshape(
        inputs.shape[0], inputs.shape[1])
    return (quantized_st, loss2d[0, 0], indices, perp2d[0, 0])
